# baseline (speedup 1.0000x reference)
import functools
import math

import jax
import jax.numpy as jnp
from jax import lax
from jax.experimental import pallas as pl
from jax.experimental.pallas import tpu as pltpu
from jax.experimental.pallas import tpu_sc as plsc

EMB = 64
SCALE = math.sqrt(EMB)
SUB = 128
CHUNK = 512


def kernel(tokens, table):
    B = tokens.shape[0] * tokens.shape[1]
    info = plsc.get_sparse_core_info()
    n_workers = info.num_cores * info.num_subcores
    b_per_w = B // n_workers
    n_chunks = b_per_w // CHUNK
    n_sub = CHUNK // SUB
    mesh = plsc.VectorSubcoreMesh(core_axis_name="c", subcore_axis_name="s")

    def body(tokens_hbm, table_hbm, out_hbm, idx_v, rows_v, sem):
        wid = lax.axis_index("s") * info.num_cores + lax.axis_index("c")
        wbase = wid * b_per_w

        def chunk_body(ci, carry):
            base = wbase + ci * CHUNK
            pltpu.sync_copy(tokens_hbm.at[pl.ds(base, CHUNK)], idx_v)
            copies = [
                pltpu.async_copy(
                    table_hbm.at[idx_v.at[pl.ds(j * SUB, SUB)]],
                    rows_v.at[pl.ds(j * SUB, SUB)],
                    sem,
                )
                for j in range(n_sub)
            ]
            for cp in copies:
                cp.wait()

            def scale_row(r, c):
                for q in range(EMB // 16):
                    sl = pl.ds(q * 16, 16)
                    rows_v[r, sl] = rows_v[r, sl] * SCALE
                return c

            lax.fori_loop(0, CHUNK, scale_row, 0, unroll=4)
            pltpu.sync_copy(rows_v, out_hbm.at[pl.ds(base, CHUNK)])
            return carry

        lax.fori_loop(0, n_chunks, chunk_body, 0)

    out = pl.kernel(
        body,
        out_type=jax.ShapeDtypeStruct((B, EMB), jnp.float32),
        mesh=mesh,
        scratch_types=[
            pltpu.VMEM((CHUNK,), jnp.int32),
            pltpu.VMEM((CHUNK, EMB), jnp.float32),
            pltpu.SemaphoreType.DMA,
        ],
        compiler_params=pltpu.CompilerParams(use_tc_tiling_on_sc=False),
    )(tokens.reshape(B), table)
    return out.reshape(tokens.shape + (EMB,))


# speedup vs baseline: 1.0895x; 1.0895x over previous
import math

import jax
import jax.numpy as jnp
from jax import lax
from jax.experimental import pallas as pl
from jax.experimental.pallas import tpu as pltpu
from jax.experimental.pallas import tpu_sc as plsc

EMB = 64
SCALE = math.sqrt(EMB)
SUB = 128
CHUNK = 256
NBUF = 4
N_SUB = CHUNK // SUB


def kernel(tokens, table):
    B = tokens.shape[0] * tokens.shape[1]
    info = plsc.get_sparse_core_info()
    n_workers = info.num_cores * info.num_subcores
    b_per_w = B // n_workers
    n_chunks = b_per_w // CHUNK
    assert n_chunks % NBUF == 0 and n_chunks >= NBUF
    mesh = plsc.VectorSubcoreMesh(core_axis_name="c", subcore_axis_name="s")

    def body(tokens_hbm, table_hbm, out_hbm, idx_v, rows, sg, ss):
        wid = lax.axis_index("s") * info.num_cores + lax.axis_index("c")
        wbase = wid * b_per_w

        pltpu.sync_copy(tokens_hbm.at[pl.ds(wbase, b_per_w)], idx_v)

        def fire(ci, b):
            for j in range(N_SUB):
                pltpu.async_copy(
                    table_hbm.at[idx_v.at[pl.ds(ci * CHUNK + j * SUB, SUB)]],
                    rows[b].at[pl.ds(j * SUB, SUB)],
                    sg[b],
                )

        def wait_gather(b):
            for j in range(N_SUB):
                pltpu.make_async_copy(
                    table_hbm.at[idx_v.at[pl.ds(j * SUB, SUB)]],
                    rows[b].at[pl.ds(j * SUB, SUB)],
                    sg[b],
                ).wait()

        def store(ci, b):
            pltpu.async_copy(rows[b], out_hbm.at[pl.ds(wbase + ci * CHUNK, CHUNK)], ss[b])

        def wait_store(b):
            pltpu.make_async_copy(rows[b], out_hbm.at[pl.ds(wbase, CHUNK)], ss[b]).wait()

        def scale(b):
            def scale_row(r, c):
                for q in range(EMB // 16):
                    sl = pl.ds(q * 16, 16)
                    rows[b][r, sl] = rows[b][r, sl] * SCALE
                return c

            lax.fori_loop(0, CHUNK, scale_row, 0, unroll=4)

        for b in range(NBUF - 1):
            fire(b, b)

        def ring_cycle(k, carry):
            for b in range(NBUF):
                ci = k * NBUF + b
                wait_gather(b)
                scale(b)
                store(ci, b)
                pb = (b - 1) % NBUF
                @pl.when(ci >= 1)
                def _():
                    wait_store(pb)
                @pl.when(ci + NBUF - 1 < n_chunks)
                def _():
                    fire(ci + NBUF - 1, pb)
            return carry

        lax.fori_loop(0, n_chunks // NBUF, ring_cycle, 0)
        wait_store((n_chunks - 1) % NBUF)

    out = pl.kernel(
        body,
        out_type=jax.ShapeDtypeStruct((B, EMB), jnp.float32),
        mesh=mesh,
        scratch_types=[
            pltpu.VMEM((b_per_w,), jnp.int32),
            [pltpu.VMEM((CHUNK, EMB), jnp.float32) for _ in range(NBUF)],
            [pltpu.SemaphoreType.DMA for _ in range(NBUF)],
            [pltpu.SemaphoreType.DMA for _ in range(NBUF)],
        ],
        compiler_params=pltpu.CompilerParams(use_tc_tiling_on_sc=False),
    )(tokens.reshape(B), table)
    return out.reshape(tokens.shape + (EMB,))


# speedup vs baseline: 1.0897x; 1.0002x over previous
import math

import jax
import jax.numpy as jnp
from jax import lax
from jax.experimental import pallas as pl
from jax.experimental.pallas import tpu as pltpu
from jax.experimental.pallas import tpu_sc as plsc

EMB = 64
SCALE = math.sqrt(EMB)
SEQ = 200
NBUF = 4
SPLITS = ((0, 128), (128, 72))


def kernel(tokens, table):
    nbatch, seq = tokens.shape
    assert seq == SEQ
    B = nbatch * seq
    info = plsc.get_sparse_core_info()
    n_workers = info.num_cores * info.num_subcores
    rows_per_w = nbatch // n_workers
    toks_per_w = rows_per_w * seq
    mesh = plsc.VectorSubcoreMesh(core_axis_name="c", subcore_axis_name="s")

    def body(tokens_hbm, table_hbm, out_hbm, idx_v, rows, sg, ss):
        wid = lax.axis_index("s") * info.num_cores + lax.axis_index("c")
        wrow0 = wid * rows_per_w

        pltpu.sync_copy(tokens_hbm.at[pl.ds(wid * toks_per_w, toks_per_w)], idx_v)

        def fire(ci, b):
            for (off, n) in SPLITS:
                pltpu.async_copy(
                    table_hbm.at[idx_v.at[pl.ds(ci * SEQ + off, n)]],
                    rows[b].at[pl.ds(off, n)],
                    sg[b],
                )

        def wait_gather(b):
            for (off, n) in SPLITS:
                pltpu.make_async_copy(
                    table_hbm.at[idx_v.at[pl.ds(off, n)]],
                    rows[b].at[pl.ds(off, n)],
                    sg[b],
                ).wait()

        def store(ci, b):
            pltpu.async_copy(rows[b], out_hbm.at[wrow0 + ci], ss[b])

        def wait_store(b):
            pltpu.make_async_copy(rows[b], out_hbm.at[wrow0], ss[b]).wait()

        def scale(b):
            def scale_row(r, c):
                for q in range(EMB // 16):
                    sl = pl.ds(q * 16, 16)
                    rows[b][r, sl] = rows[b][r, sl] * SCALE
                return c

            lax.fori_loop(0, SEQ, scale_row, 0, unroll=8)

        for b in range(NBUF - 1):
            fire(b, b)

        def ring_cycle(k, carry):
            for b in range(NBUF):
                ci = k * NBUF + b
                wait_gather(b)
                scale(b)
                store(ci, b)
                pb = (b - 1) % NBUF
                @pl.when(ci >= 1)
                def _():
                    wait_store(pb)
                @pl.when(ci + NBUF - 1 < rows_per_w)
                def _():
                    fire(ci + NBUF - 1, pb)
            return carry

        lax.fori_loop(0, rows_per_w // NBUF, ring_cycle, 0)
        wait_store((rows_per_w - 1) % NBUF)

    return pl.kernel(
        body,
        out_type=jax.ShapeDtypeStruct((nbatch, seq, EMB), jnp.float32),
        mesh=mesh,
        scratch_types=[
            pltpu.VMEM((toks_per_w,), jnp.int32),
            [pltpu.VMEM((SEQ, EMB), jnp.float32) for _ in range(NBUF)],
            [pltpu.SemaphoreType.DMA for _ in range(NBUF)],
            [pltpu.SemaphoreType.DMA for _ in range(NBUF)],
        ],
        compiler_params=pltpu.CompilerParams(use_tc_tiling_on_sc=False),
    )(tokens.reshape(B), table)
